# Initial kernel scaffold; baseline (speedup 1.0000x reference)
import jax
import jax.numpy as jnp
from jax.experimental import pallas as pl


def kernel(x, expert_queries, temperature):
    raise NotImplementedError("write your pallas kernel here")


# speedup vs baseline: 5.5998x; 5.5998x over previous
import jax
import jax.numpy as jnp
from jax.experimental import pallas as pl
from jax.experimental.pallas import tpu as pltpu

_NE = 16
_TOPK = 2
_AUXW = 0.01


def _gate_kernel(x_ref, w_ref, out_ref, aux_ref, acc_ref, *, n_rows):
    i = pl.program_id(0)
    nb = pl.num_programs(0)

    x = x_ref[...]
    w = w_ref[...]
    logits = jnp.dot(x, w, preferred_element_type=jnp.float32)

    m = jnp.max(logits, axis=-1, keepdims=True)
    e = jnp.exp(logits - m)
    s = jnp.sum(e, axis=-1, keepdims=True)
    gate = e / s
    out_ref[...] = gate

    idx = jax.lax.broadcasted_iota(jnp.int32, gate.shape, 1)
    m1 = jnp.max(gate, axis=-1, keepdims=True)
    i1 = jnp.min(jnp.where(gate == m1, idx, _NE), axis=-1, keepdims=True)
    mask1 = idx == i1
    gate2 = jnp.where(mask1, -1.0, gate)
    m2 = jnp.max(gate2, axis=-1, keepdims=True)
    i2 = jnp.min(jnp.where(gate2 == m2, idx, _NE), axis=-1, keepdims=True)
    mask = jnp.logical_or(mask1, idx == i2).astype(jnp.float32)

    gsum = jnp.sum(gate, axis=0, keepdims=True)
    csum = jnp.sum(mask, axis=0, keepdims=True)
    part = jnp.concatenate([gsum, csum], axis=0)

    @pl.when(i == 0)
    def _():
        acc_ref[...] = part

    @pl.when(i > 0)
    def _():
        acc_ref[...] = acc_ref[...] + part

    @pl.when(i == nb - 1)
    def _():
        avg = acc_ref[0:1, :] * (1.0 / n_rows)
        load = acc_ref[1:2, :] * (1.0 / (_TOPK * n_rows))
        aux_ref[0, 0] = _AUXW * _NE * jnp.sum(avg * load)


def kernel(x, expert_queries, temperature):
    B, H, W, dim = x.shape
    n = B * H * W
    x_flat = x.reshape(n, dim)
    wt = (expert_queries / temperature).T

    rows = 2048
    grid = n // rows

    import functools
    gate_flat, aux = pl.pallas_call(
        functools.partial(_gate_kernel, n_rows=n),
        grid=(grid,),
        in_specs=[
            pl.BlockSpec((rows, dim), lambda i: (i, 0)),
            pl.BlockSpec((dim, _NE), lambda i: (0, 0)),
        ],
        out_specs=[
            pl.BlockSpec((rows, _NE), lambda i: (i, 0)),
            pl.BlockSpec(memory_space=pltpu.SMEM),
        ],
        out_shape=[
            jax.ShapeDtypeStruct((n, _NE), jnp.float32),
            jax.ShapeDtypeStruct((1, 1), jnp.float32),
        ],
        scratch_shapes=[pltpu.VMEM((2, _NE), jnp.float32)],
    )(x_flat, wt)

    return gate_flat.reshape(B, H, W, _NE), aux[0, 0]


# speedup vs baseline: 8.4441x; 1.5079x over previous
import jax
import jax.numpy as jnp
from jax.experimental import pallas as pl
from jax.experimental.pallas import tpu as pltpu

_NE = 16
_TOPK = 2
_AUXW = 0.01


def _gate_kernel(x_ref, w_ref, out_ref, aux_ref, acc_ref, *, n_rows):
    i = pl.program_id(0)
    nb = pl.num_programs(0)

    x = x_ref[...]
    w = w_ref[...]
    logits = jnp.dot(x, w, preferred_element_type=jnp.float32)

    m = jnp.max(logits, axis=-1, keepdims=True)
    e = jnp.exp(logits - m)
    s = jnp.sum(e, axis=-1, keepdims=True)
    gate = e / s
    out_ref[...] = gate

    l2 = jnp.where(logits == m, -jnp.inf, logits)
    m2 = jnp.max(l2, axis=-1, keepdims=True)
    mask = (logits >= m2).astype(jnp.float32)

    gsum = jnp.sum(gate, axis=0, keepdims=True)
    csum = jnp.sum(mask, axis=0, keepdims=True)
    part = jnp.concatenate([gsum, csum], axis=0)

    @pl.when(i == 0)
    def _():
        acc_ref[...] = part

    @pl.when(i > 0)
    def _():
        acc_ref[...] = acc_ref[...] + part

    @pl.when(i == nb - 1)
    def _():
        avg = acc_ref[0:1, :] * (1.0 / n_rows)
        load = acc_ref[1:2, :] * (1.0 / (_TOPK * n_rows))
        aux_ref[0, 0] = _AUXW * _NE * jnp.sum(avg * load)


def kernel(x, expert_queries, temperature):
    B, H, W, dim = x.shape
    n = B * H * W
    x_flat = x.reshape(n, dim)
    wt = (expert_queries / temperature).T

    rows = 2048
    grid = n // rows

    import functools
    gate_flat, aux = pl.pallas_call(
        functools.partial(_gate_kernel, n_rows=n),
        grid=(grid,),
        in_specs=[
            pl.BlockSpec((rows, dim), lambda i: (i, 0)),
            pl.BlockSpec((dim, _NE), lambda i: (0, 0)),
        ],
        out_specs=[
            pl.BlockSpec((rows, _NE), lambda i: (i, 0)),
            pl.BlockSpec(memory_space=pltpu.SMEM),
        ],
        out_shape=[
            jax.ShapeDtypeStruct((n, _NE), jnp.float32),
            jax.ShapeDtypeStruct((1, 1), jnp.float32),
        ],
        scratch_shapes=[pltpu.VMEM((2, _NE), jnp.float32)],
    )(x_flat, wt)

    return gate_flat.reshape(B, H, W, _NE), aux[0, 0]


# speedup vs baseline: 10.2418x; 1.2129x over previous
import jax
import jax.numpy as jnp
from jax.experimental import pallas as pl
from jax.experimental.pallas import tpu as pltpu

_NE = 16
_TOPK = 2
_AUXW = 0.01


def _gate_kernel(x_ref, w_ref, out_ref, aux_ref, acc_ref, *, n_rows):
    i = pl.program_id(0)
    nb = pl.num_programs(0)

    x = x_ref[...]
    w = w_ref[...]
    logits = jnp.dot(x, w, preferred_element_type=jnp.float32)

    m = jnp.max(logits, axis=-1, keepdims=True)
    e = jnp.exp(logits - m)
    s = jnp.sum(e, axis=-1, keepdims=True)
    gate = e / s
    out_ref[...] = gate

    l2 = jnp.where(logits == m, -jnp.inf, logits)
    m2 = jnp.max(l2, axis=-1, keepdims=True)
    mask = (logits >= m2).astype(jnp.float32)

    gsum = jnp.sum(gate, axis=0, keepdims=True)
    csum = jnp.sum(mask, axis=0, keepdims=True)
    part = jnp.concatenate([gsum, csum], axis=0)

    @pl.when(i == 0)
    def _():
        acc_ref[...] = part

    @pl.when(i > 0)
    def _():
        acc_ref[...] = acc_ref[...] + part

    @pl.when(i == nb - 1)
    def _():
        avg = acc_ref[0:1, :] * (1.0 / n_rows)
        load = acc_ref[1:2, :] * (1.0 / (_TOPK * n_rows))
        aux_ref[0, 0] = _AUXW * _NE * jnp.sum(avg * load)


def kernel(x, expert_queries, temperature):
    B, H, W, dim = x.shape
    n = B * H * W
    x_flat = x.reshape(n, dim)
    wt = (expert_queries / temperature).T

    rows = 8192
    grid = n // rows

    import functools
    gate_flat, aux = pl.pallas_call(
        functools.partial(_gate_kernel, n_rows=n),
        grid=(grid,),
        in_specs=[
            pl.BlockSpec((rows, dim), lambda i: (i, 0)),
            pl.BlockSpec((dim, _NE), lambda i: (0, 0)),
        ],
        out_specs=[
            pl.BlockSpec((rows, _NE), lambda i: (i, 0)),
            pl.BlockSpec(memory_space=pltpu.SMEM),
        ],
        out_shape=[
            jax.ShapeDtypeStruct((n, _NE), jnp.float32),
            jax.ShapeDtypeStruct((1, 1), jnp.float32),
        ],
        scratch_shapes=[pltpu.VMEM((2, _NE), jnp.float32)],
    )(x_flat, wt)

    return gate_flat.reshape(B, H, W, _NE), aux[0, 0]
